# baseline (speedup 1.0000x reference)
import jax
import jax.numpy as jnp
from jax import lax
from jax.experimental import pallas as pl
from jax.experimental.pallas import tpu as pltpu
from jax.experimental.pallas import tpu_sc as plsc

N_ROWS = 1000000
N_FACTORS = 32
BATCH = 16384
NUM_CORES = 2
NUM_SUBCORES = 16
NUM_WORKERS = NUM_CORES * NUM_SUBCORES
PAIRS_PER_WORKER = BATCH // NUM_WORKERS
CHUNK = 128
NUM_CHUNKS = PAIRS_PER_WORKER // CHUNK
LANES = 16
GROUPS_PER_CHUNK = CHUNK // LANES
WIDE = 128
PACK = WIDE // N_FACTORS


def _body(data_hbm, uf_hbm, if_hbm, out_hbm,
          data_v, uidx_v, iidx_v, ucol_v, icol_v, urows_v, irows_v,
          out_v, sem):
    wid = lax.axis_index("s") * NUM_CORES + lax.axis_index("c")
    base = wid * PAIRS_PER_WORKER

    pltpu.sync_copy(data_hbm.at[pl.ds(base, PAIRS_PER_WORKER), :], data_v)

    lane = lax.iota(jnp.int32, 16)
    zeros = jnp.zeros((16,), jnp.int32)
    ones = jnp.ones((16,), jnp.int32)
    for g in range(PAIRS_PER_WORKER // LANES):
        rows = g * LANES + lane
        u_ids = plsc.load_gather(data_v, [rows, zeros])
        i_ids = plsc.load_gather(data_v, [rows, ones])
        c, o = divmod(g * LANES, CHUNK)
        uidx_v[c, pl.ds(o, LANES)] = u_ids // PACK
        iidx_v[c, pl.ds(o, LANES)] = i_ids // PACK
        ucol_v[c, pl.ds(o, LANES)] = (u_ids % PACK) * N_FACTORS
        icol_v[c, pl.ds(o, LANES)] = (i_ids % PACK) * N_FACTORS

    for k in range(NUM_CHUNKS):
        cu = pltpu.async_copy(uf_hbm.at[uidx_v.at[k]], urows_v, sem)
        ci = pltpu.async_copy(if_hbm.at[iidx_v.at[k]], irows_v, sem)
        cu.wait()
        ci.wait()

        def group_body(g, carry, k=k):
            row_in = g * LANES + lane
            ucol = ucol_v[k, pl.ds(g * LANES, LANES)]
            icol = icol_v[k, pl.ds(g * LANES, LANES)]
            acc = jnp.zeros((16,), jnp.float32)
            for j in range(N_FACTORS):
                uu = plsc.load_gather(urows_v, [row_in, ucol + j])
                vv = plsc.load_gather(irows_v, [row_in, icol + j])
                acc = acc + uu * vv
            out_v[pl.ds(k * CHUNK + g * LANES, LANES)] = acc
            return carry

        lax.fori_loop(0, GROUPS_PER_CHUNK, group_body, 0)

    pltpu.sync_copy(out_v, out_hbm.at[pl.ds(base, PAIRS_PER_WORKER)])


@jax.jit
def kernel(data, user_factors, item_factors):
    mesh = plsc.VectorSubcoreMesh(
        core_axis_name="c", subcore_axis_name="s",
        num_cores=NUM_CORES, num_subcores=NUM_SUBCORES)
    run = pl.kernel(
        _body,
        jax.ShapeDtypeStruct((BATCH,), jnp.float32),
        mesh=mesh,
        compiler_params=pltpu.CompilerParams(needs_layout_passes=False),
        scratch_types=[
            pltpu.VMEM((PAIRS_PER_WORKER, 2), jnp.int32),
            pltpu.VMEM((NUM_CHUNKS, CHUNK), jnp.int32),
            pltpu.VMEM((NUM_CHUNKS, CHUNK), jnp.int32),
            pltpu.VMEM((NUM_CHUNKS, CHUNK), jnp.int32),
            pltpu.VMEM((NUM_CHUNKS, CHUNK), jnp.int32),
            pltpu.VMEM((CHUNK, WIDE), jnp.float32),
            pltpu.VMEM((CHUNK, WIDE), jnp.float32),
            pltpu.VMEM((PAIRS_PER_WORKER,), jnp.float32),
            pltpu.SemaphoreType.DMA,
        ],
    )
    uf = user_factors.reshape(N_ROWS // PACK, WIDE)
    vf = item_factors.reshape(N_ROWS // PACK, WIDE)
    return run(data.astype(jnp.int32), uf, vf)


# speedup vs baseline: 3.6406x; 3.6406x over previous
import jax
import jax.numpy as jnp
from jax import lax
from jax.experimental import pallas as pl
from jax.experimental.pallas import tpu as pltpu
from jax.experimental.pallas import tpu_sc as plsc

N_ROWS = 1000000
N_FACTORS = 32
BATCH = 16384
NUM_CORES = 2
NUM_SUBCORES = 16
NUM_WORKERS = NUM_CORES * NUM_SUBCORES
PAIRS_PER_WORKER = BATCH // NUM_WORKERS
LANES = 16
CHUNK = 16
NUM_CHUNKS = PAIRS_PER_WORKER // CHUNK
TILE_W = 128


def _fetch_chunk(table_hbm, ids_vec, blk_v, sem):
    col0_vec = (ids_vec >> 7) << 7
    copies = []
    for p in range(CHUNK):
        c0 = pl.multiple_of(col0_vec[p], TILE_W)
        copies.append(pltpu.async_copy(
            table_hbm.at[:, pl.ds(c0, TILE_W)], blk_v.at[p], sem))
    return copies


def _extract(blk_v, ids_vec, lane):
    col = ids_vec & 127
    outs = []
    for p in range(CHUNK):
        cv = jnp.full((16,), 1, jnp.int32) * col[p]
        pv = jnp.full((16,), p, jnp.int32)
        lo = plsc.load_gather(blk_v, [pv, lane, cv])
        hi = plsc.load_gather(blk_v, [pv, lane + LANES, cv])
        outs.append((lo, hi))
    return outs


def _body(dataT_hbm, uf_hbm, if_hbm, out_hbm,
          ids_v, blk_v, uval_v, out_v, sem):
    wid = lax.axis_index("s") * NUM_CORES + lax.axis_index("c")
    base = wid * PAIRS_PER_WORKER
    lane = lax.iota(jnp.int32, 16)

    pltpu.sync_copy(dataT_hbm.at[:, pl.ds(base, PAIRS_PER_WORKER)], ids_v)

    def user_chunk(k, carry):
        pair0 = k * CHUNK
        u_vec = ids_v[0, pl.ds(pair0, CHUNK)]
        for c in _fetch_chunk(uf_hbm, u_vec, blk_v, sem):
            c.wait()
        vals = _extract(blk_v, u_vec, lane)
        for p, (lo, hi) in enumerate(vals):
            row = k * 4 + p // 4
            col = (p % 4) * N_FACTORS
            uval_v[row, pl.ds(col, LANES)] = lo
            uval_v[row, pl.ds(col + LANES, LANES)] = hi
        return carry

    lax.fori_loop(0, NUM_CHUNKS, user_chunk, 0)

    def item_chunk(k, carry):
        pair0 = k * CHUNK
        i_vec = ids_v[1, pl.ds(pair0, CHUNK)]
        for c in _fetch_chunk(if_hbm, i_vec, blk_v, sem):
            c.wait()
        vals = _extract(blk_v, i_vec, lane)
        res = jnp.zeros((16,), jnp.float32)
        for p, (lo, hi) in enumerate(vals):
            row = k * 4 + p // 4
            col = (p % 4) * N_FACTORS
            ulo = uval_v[row, pl.ds(col, LANES)]
            uhi = uval_v[row, pl.ds(col + LANES, LANES)]
            prod = lo * ulo + hi * uhi
            s = lax.reduce_sum(prod, axes=(0,))
            res = jnp.where(lane == p, s, res)
        out_v[pl.ds(pair0, CHUNK)] = res
        return carry

    lax.fori_loop(0, NUM_CHUNKS, item_chunk, 0)

    pltpu.sync_copy(out_v, out_hbm.at[pl.ds(base, PAIRS_PER_WORKER)])


@jax.jit
def kernel(data, user_factors, item_factors):
    mesh = plsc.VectorSubcoreMesh(
        core_axis_name="c", subcore_axis_name="s",
        num_cores=NUM_CORES, num_subcores=NUM_SUBCORES)
    run = pl.kernel(
        _body,
        jax.ShapeDtypeStruct((BATCH,), jnp.float32),
        mesh=mesh,
        compiler_params=pltpu.CompilerParams(needs_layout_passes=False),
        scratch_types=[
            pltpu.VMEM((2, PAIRS_PER_WORKER), jnp.int32),
            pltpu.VMEM((CHUNK, N_FACTORS, TILE_W), jnp.float32),
            pltpu.VMEM((PAIRS_PER_WORKER // 4, 4 * N_FACTORS), jnp.float32),
            pltpu.VMEM((PAIRS_PER_WORKER,), jnp.float32),
            pltpu.SemaphoreType.DMA,
        ],
    )
    return run(data.T.astype(jnp.int32), user_factors.T, item_factors.T)


# speedup vs baseline: 4.4163x; 1.2131x over previous
import jax
import jax.numpy as jnp
from jax import lax
from jax.experimental import pallas as pl
from jax.experimental.pallas import tpu as pltpu
from jax.experimental.pallas import tpu_sc as plsc

N_ROWS = 1000000
N_FACTORS = 32
BATCH = 16384
NUM_CORES = 2
NUM_SUBCORES = 16
NUM_WORKERS = NUM_CORES * NUM_SUBCORES
PAIRS_PER_WORKER = BATCH // NUM_WORKERS
LANES = 16
TILE_W = 128
N_SLOTS = 8
GROUPS = PAIRS_PER_WORKER // LANES


def _fire(table_hbm, col0, slot_ref, sem):
    c0 = pl.multiple_of(col0, TILE_W)
    pltpu.async_copy(table_hbm.at[:, pl.ds(c0, TILE_W)], slot_ref, sem)


def _drain(table_hbm, slot_ref, sem):
    pltpu.make_async_copy(
        table_hbm.at[:, pl.ds(0, TILE_W)], slot_ref, sem).wait()


def _body(dataT_hbm, uf_hbm, if_hbm, out_hbm,
          ids_v, ublk_v, iblk_v, out_v,
          sem0, sem1, sem2, sem3, sem4, sem5, sem6, sem7):
    sems = [sem0, sem1, sem2, sem3, sem4, sem5, sem6, sem7]
    wid = lax.axis_index("s") * NUM_CORES + lax.axis_index("c")
    base = wid * PAIRS_PER_WORKER
    lane = lax.iota(jnp.int32, LANES)

    pltpu.sync_copy(dataT_hbm.at[:, pl.ds(base, PAIRS_PER_WORKER)], ids_v)

    u16_0 = ids_v[0, pl.ds(0, LANES)]
    i16_0 = ids_v[1, pl.ds(0, LANES)]
    uc0_0 = (u16_0 >> 7) << 7
    ic0_0 = (i16_0 >> 7) << 7
    for s in range(N_SLOTS):
        _fire(uf_hbm, uc0_0[s], ublk_v.at[s], sems[s])
        _fire(if_hbm, ic0_0[s], iblk_v.at[s], sems[s])

    def group(g, carry):
        pair0 = g * LANES
        u16 = ids_v[0, pl.ds(pair0, LANES)]
        i16 = ids_v[1, pl.ds(pair0, LANES)]
        gn = (g + 1) & (GROUPS - 1)
        u16n = ids_v[0, pl.ds(gn * LANES, LANES)]
        i16n = ids_v[1, pl.ds(gn * LANES, LANES)]
        cu = u16 & 127
        ci = i16 & 127
        uc0 = (u16 >> 7) << 7
        ic0 = (i16 >> 7) << 7
        uc0n = (u16n >> 7) << 7
        ic0n = (i16n >> 7) << 7
        one = jnp.full((LANES,), 1, jnp.int32)
        res = jnp.zeros((LANES,), jnp.float32)
        for j in range(LANES):
            s = j % N_SLOTS
            sv = one * s
            _drain(uf_hbm, ublk_v.at[s], sems[s])
            _drain(if_hbm, iblk_v.at[s], sems[s])
            cuv = one * cu[j]
            civ = one * ci[j]
            ulo = plsc.load_gather(ublk_v, [sv, lane, cuv])
            uhi = plsc.load_gather(ublk_v, [sv, lane + LANES, cuv])
            ilo = plsc.load_gather(iblk_v, [sv, lane, civ])
            ihi = plsc.load_gather(iblk_v, [sv, lane + LANES, civ])
            prod = ulo * ilo + uhi * ihi
            d = lax.reduce_sum(prod, axes=(0,))
            res = jnp.where(lane == j, d, res)
            if j < N_SLOTS:
                _fire(uf_hbm, uc0[j + N_SLOTS], ublk_v.at[s], sems[s])
                _fire(if_hbm, ic0[j + N_SLOTS], iblk_v.at[s], sems[s])
            else:
                _fire(uf_hbm, uc0n[j - N_SLOTS], ublk_v.at[s], sems[s])
                _fire(if_hbm, ic0n[j - N_SLOTS], iblk_v.at[s], sems[s])
        out_v[pl.ds(pair0, LANES)] = res
        return carry

    lax.fori_loop(0, GROUPS, group, 0)

    for s in range(N_SLOTS):
        _drain(uf_hbm, ublk_v.at[s], sems[s])
        _drain(if_hbm, iblk_v.at[s], sems[s])

    pltpu.sync_copy(out_v, out_hbm.at[pl.ds(base, PAIRS_PER_WORKER)])


@jax.jit
def kernel(data, user_factors, item_factors):
    mesh = plsc.VectorSubcoreMesh(
        core_axis_name="c", subcore_axis_name="s",
        num_cores=NUM_CORES, num_subcores=NUM_SUBCORES)
    run = pl.kernel(
        _body,
        jax.ShapeDtypeStruct((BATCH,), jnp.float32),
        mesh=mesh,
        compiler_params=pltpu.CompilerParams(needs_layout_passes=False),
        scratch_types=[
            pltpu.VMEM((2, PAIRS_PER_WORKER), jnp.int32),
            pltpu.VMEM((N_SLOTS, N_FACTORS, TILE_W), jnp.float32),
            pltpu.VMEM((N_SLOTS, N_FACTORS, TILE_W), jnp.float32),
            pltpu.VMEM((PAIRS_PER_WORKER,), jnp.float32),
        ] + [pltpu.SemaphoreType.DMA] * N_SLOTS,
    )
    return run(data.T.astype(jnp.int32), user_factors.T, item_factors.T)
